# baseline (speedup 1.0000x reference)
import functools

import jax
import jax.numpy as jnp
from jax.experimental import pallas as pl
from jax.experimental.pallas import tpu as pltpu


@functools.lru_cache(maxsize=None)
def _build(seq_len: int, nhid: int):
    blk = 256
    assert seq_len % blk == 0
    grid = seq_len // blk

    def body(in_ref, out_ref):
        out_ref[...] = in_ref[...]

    return pl.pallas_call(
        body,
        grid=(grid,),
        in_specs=[pl.BlockSpec((blk, nhid), lambda i: (i, 0))],
        out_specs=pl.BlockSpec((blk, nhid), lambda i: (i, 0)),
        out_shape=jax.ShapeDtypeStruct((seq_len, nhid), jnp.float32),
    )


def kernel(x, pe):
    seq_len = x.shape[1]
    nhid = pe.shape[2]
    out2d = _build(seq_len, nhid)(pe.reshape(pe.shape[1], nhid))
    return out2d.reshape(1, seq_len, nhid)


# speedup vs baseline: 1.4367x; 1.4367x over previous
import functools

import jax
import jax.numpy as jnp
from jax.experimental import pallas as pl
from jax.experimental.pallas import tpu as pltpu


@functools.lru_cache(maxsize=None)
def _build(seq_len: int, nhid: int):
    blk = 1024
    assert seq_len % blk == 0
    grid = seq_len // blk

    def body(in_ref, out_ref):
        out_ref[...] = in_ref[...]

    return pl.pallas_call(
        body,
        grid=(grid,),
        in_specs=[pl.BlockSpec((blk, nhid), lambda i: (i, 0))],
        out_specs=pl.BlockSpec((blk, nhid), lambda i: (i, 0)),
        out_shape=jax.ShapeDtypeStruct((seq_len, nhid), jnp.float32),
    )


def kernel(x, pe):
    seq_len = x.shape[1]
    nhid = pe.shape[2]
    out2d = _build(seq_len, nhid)(pe.reshape(pe.shape[1], nhid))
    return out2d.reshape(1, seq_len, nhid)


# speedup vs baseline: 1.6169x; 1.1254x over previous
import functools

import jax
import jax.numpy as jnp
from jax.experimental import pallas as pl
from jax.experimental.pallas import tpu as pltpu


@functools.lru_cache(maxsize=None)
def _build(seq_len: int, nhid: int):
    blk = 2048
    assert seq_len % blk == 0
    grid = seq_len // blk

    def body(in_ref, out_ref):
        out_ref[...] = in_ref[...]

    return pl.pallas_call(
        body,
        grid=(grid,),
        in_specs=[pl.BlockSpec((blk, nhid), lambda i: (i, 0))],
        out_specs=pl.BlockSpec((blk, nhid), lambda i: (i, 0)),
        out_shape=jax.ShapeDtypeStruct((seq_len, nhid), jnp.float32),
    )


def kernel(x, pe):
    seq_len = x.shape[1]
    nhid = pe.shape[2]
    out2d = _build(seq_len, nhid)(pe.reshape(pe.shape[1], nhid))
    return out2d.reshape(1, seq_len, nhid)
